# Initial kernel scaffold; baseline (speedup 1.0000x reference)
import jax
import jax.numpy as jnp
from jax.experimental import pallas as pl


def kernel(nodes, edge_index, edge_attr, valid, W1, b1, W2, b2, Wg, Ww, bw, Wf, bf):
    raise NotImplementedError("write your pallas kernel here")


# speedup vs baseline: 19.8507x; 19.8507x over previous
import functools

import jax
import jax.numpy as jnp
from jax import lax
from jax.experimental import pallas as pl
from jax.experimental.pallas import tpu as pltpu
from jax.experimental.pallas import tpu_sc as plsc

N = 10000
L = 128
E = 320000
H = 16

EC = 12800
NSTEP_E = E // EC
NB = 1000
NSTEP_N = N // NB

NPAD = 10240
EPW = E // 32
BATCH = 80
NBATCH = EPW // BATCH
ROWS_PER_SUB = NPAD // 16


def _f32(x):
    return jnp.asarray(x, jnp.float32)


def _edge_scan_body(eT_ref, W1_ref, b1_ref, W2_ref, b2_ref, Wg_ref,
                    Ww_ref, bw_ref, w0_ref, carry_ref):
    i = pl.program_id(0)

    eT = eT_ref[...]
    ea = jnp.dot(W1_ref[...], eT, preferred_element_type=jnp.float32)
    ea = ea + b1_ref[...]
    ea = 0.5 * ea * (1.0 + lax.erf(ea * 0.7071067811865476))
    ea = jnp.dot(W2_ref[...], ea, preferred_element_type=jnp.float32)
    ea = jnp.maximum(ea + b2_ref[...], 0.0)
    hg = jnp.dot(Wg_ref[...], ea, preferred_element_type=jnp.float32)
    hidden = hg[:H, :]
    gate = hg[H:, :]
    z = jax.nn.sigmoid(gate)
    a = 1.0 - z
    g = jnp.where(hidden >= 0, hidden + 0.5, jax.nn.sigmoid(hidden))
    b = z * g

    lane = lax.broadcasted_iota(jnp.int32, (H, EC), 1)
    s = 1
    while s < EC:
        a_sh = pltpu.roll(a, s, 1)
        b_sh = pltpu.roll(b, s, 1)
        ok = lane >= s
        a_sh = jnp.where(ok, a_sh, 1.0)
        b_sh = jnp.where(ok, b_sh, 0.0)
        b = a * b_sh + b
        a = a * a_sh
        s *= 2

    @pl.when(i == 0)
    def _():
        carry_ref[...] = jnp.zeros_like(carry_ref)

    h = a * carry_ref[:, 0:1] + b
    carry_ref[:, 0:1] = h[:, EC - 1:EC]

    w0 = jnp.dot(Ww_ref[...], h, preferred_element_type=jnp.float32)
    w0_ref[...] = w0[0:1, :] + bw_ref[...]


def _edge_scan(eTpad, W1pad, b1, W2, b2, Wg, Wwpad, bw):
    return pl.pallas_call(
        _edge_scan_body,
        grid=(NSTEP_E,),
        in_specs=[
            pl.BlockSpec((24, EC), lambda i: (0, i)),
            pl.BlockSpec((16, 24), lambda i: (0, 0)),
            pl.BlockSpec((16, 1), lambda i: (0, 0)),
            pl.BlockSpec((16, 16), lambda i: (0, 0)),
            pl.BlockSpec((16, 1), lambda i: (0, 0)),
            pl.BlockSpec((32, 16), lambda i: (0, 0)),
            pl.BlockSpec((8, 16), lambda i: (0, 0)),
            pl.BlockSpec((1, 1), lambda i: (0, 0)),
        ],
        out_specs=pl.BlockSpec((1, EC), lambda i: (0, i)),
        out_shape=jax.ShapeDtypeStruct((1, E), jnp.float32),
        scratch_shapes=[pltpu.VMEM((16, 128), jnp.float32)],
    )(eTpad, W1pad, b1, W2, b2, Wg, Wwpad, bw)


def _prep_body(nodes_ref, vs_ref, x0_ref, mv_ref):
    vs = vs_ref[...]
    x0_ref[...] = nodes_ref[...] * vs
    mv_ref[...] = jnp.mean(vs, axis=1, keepdims=True)


def _node_prep(nodes2d, vs2d):
    return pl.pallas_call(
        _prep_body,
        grid=(NSTEP_N,),
        in_specs=[
            pl.BlockSpec((NB, L), lambda i: (i, 0)),
            pl.BlockSpec((NB, L), lambda i: (i, 0)),
        ],
        out_specs=[
            pl.BlockSpec((NB, L), lambda i: (i, 0)),
            pl.BlockSpec((NB, 1), lambda i: (i, 0)),
        ],
        out_shape=[
            jax.ShapeDtypeStruct((N, L), jnp.float32),
            jax.ShapeDtypeStruct((N, 1), jnp.float32),
        ],
    )(nodes2d, vs2d)


_MESH = plsc.VectorSubcoreMesh(core_axis_name="c", subcore_axis_name="s")


def _wid():
    c = lax.axis_index("c")
    s = lax.axis_index("s")
    return c * 16 + s, c, s


def _sc_w_body(mv_hbm, w0_hbm, src_hbm, w_hbm, pmax_hbm,
               mv_v, w0_v, src_v, w_v, max_v):
    wid, c, s = _wid()
    base = wid * EPW
    pltpu.sync_copy(mv_hbm, mv_v)
    pltpu.sync_copy(w0_hbm.at[pl.ds(base, EPW)], w0_v)
    pltpu.sync_copy(src_hbm.at[pl.ds(base, EPW)], src_v)

    def grp(g, m):
        o = g * 16
        idx = src_v[pl.ds(o, 16)]
        w0g = w0_v[pl.ds(o, 16)]
        mvv = plsc.load_gather(mv_v, [idx])
        wg = mvv * w0g
        w_v[pl.ds(o, 16)] = wg
        return jnp.maximum(m, wg)

    m = lax.fori_loop(0, EPW // 16, grp,
                      jnp.full((16,), -jnp.inf, jnp.float32))
    max_v[...] = m
    pltpu.sync_copy(w_v, w_hbm.at[pl.ds(base, EPW)])
    pltpu.sync_copy(max_v, pmax_hbm.at[wid])


def _sc_w(mv, w0, src):
    kfn = pl.kernel(
        _sc_w_body,
        out_type=[
            jax.ShapeDtypeStruct((E,), jnp.float32),
            jax.ShapeDtypeStruct((32, 16), jnp.float32),
        ],
        mesh=_MESH,
        compiler_params=pltpu.CompilerParams(needs_layout_passes=False),
        scratch_types=[
            pltpu.VMEM((N,), jnp.float32),
            pltpu.VMEM((EPW,), jnp.float32),
            pltpu.VMEM((EPW,), jnp.int32),
            pltpu.VMEM((EPW,), jnp.float32),
            pltpu.VMEM((16,), jnp.float32),
        ],
    )
    return kfn(mv, w0, src)


def _global_max(pmax_v):
    m = pmax_v[0, :]
    for j in range(1, 32):
        m = jnp.maximum(m, pmax_v[j, :])
    return lax.reduce_max(m, (0,))


def _sc_sums_body(w_hbm, dst_hbm, pmax_hbm, psums_hbm,
                  w_v, dst_v, e_v, pmax_v, z_v, idx_b, sums_sh):
    wid, c, s = _wid()
    base = wid * EPW

    def zi(i, _):
        z_v[pl.ds(i * 16, 16)] = jnp.zeros((16,), jnp.float32)
        return 0
    lax.fori_loop(0, ROWS_PER_SUB // 16, zi, 0)
    pltpu.sync_copy(z_v, sums_sh.at[pl.ds(s * ROWS_PER_SUB, ROWS_PER_SUB)])

    pltpu.sync_copy(w_hbm.at[pl.ds(base, EPW)], w_v)
    pltpu.sync_copy(dst_hbm.at[pl.ds(base, EPW)], dst_v)
    pltpu.sync_copy(pmax_hbm, pmax_v)
    cmax = _global_max(pmax_v)

    plsc.subcore_barrier()

    def batch(bi, _):
        o = bi * BATCH
        for k in range(BATCH // 16):
            wg = w_v[pl.ds(o + k * 16, 16)]
            e_v[pl.ds(o + k * 16, 16)] = jnp.exp(wg - cmax)
        for k in range(BATCH // 16):
            idx_b[pl.ds(k * 16, 16)] = dst_v[pl.ds(o + k * 16, 16)]
        pltpu.sync_copy(e_v.at[pl.ds(o, BATCH)],
                        sums_sh.at[idx_b], add=True)
        return 0
    lax.fori_loop(0, NBATCH, batch, 0)

    plsc.subcore_barrier()
    pltpu.sync_copy(sums_sh.at[pl.ds(s * ROWS_PER_SUB, ROWS_PER_SUB)],
                    psums_hbm.at[pl.ds(c * NPAD + s * ROWS_PER_SUB,
                                       ROWS_PER_SUB)])


def _sc_sums(w, dst, pmax):
    kfn = pl.kernel(
        _sc_sums_body,
        out_type=jax.ShapeDtypeStruct((2 * NPAD,), jnp.float32),
        mesh=_MESH,
        compiler_params=pltpu.CompilerParams(needs_layout_passes=False),
        scratch_types=[
            pltpu.VMEM((EPW,), jnp.float32),
            pltpu.VMEM((EPW,), jnp.int32),
            pltpu.VMEM((EPW,), jnp.float32),
            pltpu.VMEM((32, 16), jnp.float32),
            pltpu.VMEM((ROWS_PER_SUB,), jnp.float32),
            pltpu.VMEM((BATCH,), jnp.int32),
            pltpu.VMEM_SHARED((NPAD,), jnp.float32),
        ],
    )
    return kfn(w, dst, pmax)


def _sc_weights_body(w_hbm, dst_hbm, pmax_hbm, psums_hbm, wt_hbm,
                     w_v, dst_v, wt_v, pmax_v, s0_v, s1_v):
    wid, c, s = _wid()
    base = wid * EPW

    pltpu.sync_copy(psums_hbm.at[pl.ds(0, NPAD)], s0_v)
    pltpu.sync_copy(psums_hbm.at[pl.ds(NPAD, NPAD)], s1_v)
    pltpu.sync_copy(pmax_hbm, pmax_v)
    cmax = _global_max(pmax_v)
    pltpu.sync_copy(w_hbm.at[pl.ds(base, EPW)], w_v)
    pltpu.sync_copy(dst_hbm.at[pl.ds(base, EPW)], dst_v)

    def si(i, _):
        o = i * 16
        s0_v[pl.ds(o, 16)] = s0_v[pl.ds(o, 16)] + s1_v[pl.ds(o, 16)]
        return 0
    lax.fori_loop(0, NPAD // 16, si, 0)

    def grp(g, _):
        o = g * 16
        wg = w_v[pl.ds(o, 16)]
        d = dst_v[pl.ds(o, 16)]
        e = jnp.exp(wg - cmax)
        sv = plsc.load_gather(s0_v, [d])
        wt_v[pl.ds(o, 16)] = e / (sv + 1e-16)
        return 0
    lax.fori_loop(0, EPW // 16, grp, 0)
    pltpu.sync_copy(wt_v, wt_hbm.at[pl.ds(base, EPW)])


def _sc_weights(w, dst, pmax, psums):
    kfn = pl.kernel(
        _sc_weights_body,
        out_type=jax.ShapeDtypeStruct((E,), jnp.float32),
        mesh=_MESH,
        compiler_params=pltpu.CompilerParams(needs_layout_passes=False),
        scratch_types=[
            pltpu.VMEM((EPW,), jnp.float32),
            pltpu.VMEM((EPW,), jnp.int32),
            pltpu.VMEM((EPW,), jnp.float32),
            pltpu.VMEM((32, 16), jnp.float32),
            pltpu.VMEM((NPAD,), jnp.float32),
            pltpu.VMEM((NPAD,), jnp.float32),
        ],
    )
    return kfn(w, dst, pmax, psums)


def _sc_spmm_body(x_hbm, src_hbm, dst_hbm, wt_hbm, out_hbm,
                  src_v, dst_v, wt_v, rows_v, z_v, idx_b, agg_sh):
    wid, c, s = _wid()
    base = wid * EPW

    def zrow(i, _):
        for k in range(8):
            z_v[i, pl.ds(k * 16, 16)] = jnp.zeros((16,), jnp.float32)
        return 0
    lax.fori_loop(0, BATCH, zrow, 0)
    for j in range(8):
        row = s * 640 + j * 80

        @pl.when(row < N)
        def _():
            pltpu.sync_copy(z_v, agg_sh.at[pl.ds(row, BATCH)])

    pltpu.sync_copy(src_hbm.at[pl.ds(base, EPW)], src_v)
    pltpu.sync_copy(dst_hbm.at[pl.ds(base, EPW)], dst_v)
    pltpu.sync_copy(wt_hbm.at[pl.ds(base, EPW)], wt_v)

    plsc.subcore_barrier()

    def batch(bi, _):
        o = bi * BATCH
        pltpu.sync_copy(x_hbm.at[src_v.at[pl.ds(o, BATCH)]], rows_v)

        def quad(k, _):
            for j in range(16):
                eidx = o + k * 16 + j
                sp = plsc.load_gather(
                    wt_v, [jnp.full((16,), eidx, jnp.int32)])
                r = k * 16 + j
                for q in range(8):
                    rows_v[r, pl.ds(q * 16, 16)] = (
                        rows_v[r, pl.ds(q * 16, 16)] * sp)
            return 0
        lax.fori_loop(0, BATCH // 16, quad, 0)

        for k in range(BATCH // 16):
            idx_b[pl.ds(k * 16, 16)] = dst_v[pl.ds(o + k * 16, 16)]
        pltpu.sync_copy(rows_v, agg_sh.at[idx_b], add=True)
        return 0
    lax.fori_loop(0, NBATCH, batch, 0)

    plsc.subcore_barrier()
    for j in range(8):
        row = s * 640 + j * 80

        @pl.when(row < N)
        def _():
            pltpu.sync_copy(agg_sh.at[pl.ds(row, BATCH)],
                            out_hbm.at[pl.ds(c * N + row, BATCH)])


def _sc_spmm(x, src, dst, wt):
    kfn = pl.kernel(
        _sc_spmm_body,
        out_type=jax.ShapeDtypeStruct((2 * N, L), jnp.float32),
        mesh=_MESH,
        compiler_params=pltpu.CompilerParams(needs_layout_passes=False),
        scratch_types=[
            pltpu.VMEM((EPW,), jnp.int32),
            pltpu.VMEM((EPW,), jnp.int32),
            pltpu.VMEM((EPW,), jnp.float32),
            pltpu.VMEM((BATCH, L), jnp.float32),
            pltpu.VMEM((BATCH, L), jnp.float32),
            pltpu.VMEM((BATCH,), jnp.int32),
            pltpu.VMEM_SHARED((N, L), jnp.float32),
        ],
    )
    return kfn(x, src, dst, wt)


def _upd_body(p_ref, x_ref, vs_ref, wf_ref, bf_ref, out_ref):
    agg = p_ref[0] + p_ref[1]
    xv = x_ref[...]
    vsv = vs_ref[...]
    nv = 1.0 - vsv
    m = jax.nn.sigmoid(wf_ref[0, 0] * agg + wf_ref[0, 1] * xv
                       + wf_ref[0, 2] * nv + bf_ref[0])
    out_ref[...] = (1.0 - m) * xv + nv * m * agg


def _update(partials, x, vs2d, Wf, bf):
    return pl.pallas_call(
        _upd_body,
        grid=(NSTEP_N,),
        in_specs=[
            pl.BlockSpec((2, NB, L), lambda i: (0, i, 0)),
            pl.BlockSpec((NB, L), lambda i: (i, 0)),
            pl.BlockSpec((NB, L), lambda i: (i, 0)),
            pl.BlockSpec(memory_space=pltpu.SMEM),
            pl.BlockSpec(memory_space=pltpu.SMEM),
        ],
        out_specs=pl.BlockSpec((NB, L), lambda i: (i, 0)),
        out_shape=jax.ShapeDtypeStruct((N, L), jnp.float32),
    )(partials, x, vs2d, Wf, bf)


def kernel(nodes, edge_index, edge_attr, valid, W1, b1, W2, b2, Wg, Ww, bw,
           Wf, bf):
    nodes2d = nodes[..., 0]
    vs2d = valid[0]
    src = edge_index[0].astype(jnp.int32)
    dst = edge_index[1].astype(jnp.int32)

    eTpad = jnp.zeros((24, E), jnp.float32).at[:18, :].set(edge_attr.T)
    W1pad = jnp.zeros((16, 24), jnp.float32).at[:, :18].set(W1)
    Wwpad = jnp.zeros((8, 16), jnp.float32).at[0:1, :].set(Ww)

    w0 = _edge_scan(eTpad, W1pad, b1.reshape(16, 1), W2, b2.reshape(16, 1),
                    Wg, Wwpad, bw.reshape(1, 1))
    x0, mv = _node_prep(nodes2d, vs2d)

    w, pmax = _sc_w(mv.reshape(N), w0.reshape(E), src)
    psums = _sc_sums(w, dst, pmax)
    wt = _sc_weights(w, dst, pmax, psums)

    x = x0
    for _ in range(3):
        part = _sc_spmm(x, src, dst, wt)
        part = part.reshape(2, N, L)
        x = _update(part, x, vs2d, Wf, bf)
    return x[0][None, :]
